# baseline (speedup 1.0000x reference)
import functools

import jax
import jax.numpy as jnp
from jax import lax
from jax.experimental import pallas as pl
from jax.experimental.pallas import tpu as pltpu
from jax.experimental.pallas import tpu_sc as plsc

NUM_CORES = 2
NUM_SUBCORES = 16
NW = NUM_CORES * NUM_SUBCORES

B = 16384
D = 64
HIDDEN = 128
CHUNK = 128
B_PER_W = B // NW
CH_PER_W = B_PER_W // CHUNK


def _sc_gather(user_table, item_table, uidx2d, iidx2d):
    mesh = plsc.VectorSubcoreMesh(core_axis_name="c", subcore_axis_name="s")

    @functools.partial(
        pl.kernel,
        out_type=(
            jax.ShapeDtypeStruct((B, D), jnp.float32),
            jax.ShapeDtypeStruct((B, D), jnp.float32),
        ),
        mesh=mesh,
        compiler_params=pltpu.CompilerParams(use_tc_tiling_on_sc=False),
        scratch_types=[
            pltpu.VMEM((CH_PER_W, CHUNK), jnp.int32),
            pltpu.VMEM((CH_PER_W, CHUNK), jnp.int32),
            pltpu.VMEM((B_PER_W, D), jnp.float32),
            pltpu.VMEM((B_PER_W, D), jnp.float32),
            pltpu.SemaphoreType.DMA,
            pltpu.SemaphoreType.DMA,
        ],
    )
    def gather_kernel(ut_hbm, it_hbm, uix_hbm, iix_hbm, uout_hbm, iout_hbm,
                      uix_v, iix_v, urows, irows, usem, isem):
        wid = lax.axis_index("s") * NUM_CORES + lax.axis_index("c")
        row0 = wid * CH_PER_W
        pltpu.sync_copy(uix_hbm.at[pl.ds(row0, CH_PER_W)], uix_v)
        pltpu.sync_copy(iix_hbm.at[pl.ds(row0, CH_PER_W)], iix_v)
        copies = []
        for j in range(CH_PER_W):
            dst = pl.ds(j * CHUNK, CHUNK)
            copies.append(
                pltpu.async_copy(ut_hbm.at[uix_v.at[j]], urows.at[dst], usem))
            copies.append(
                pltpu.async_copy(it_hbm.at[iix_v.at[j]], irows.at[dst], isem))
        for c in copies:
            c.wait()
        base = wid * B_PER_W
        pltpu.sync_copy(urows, uout_hbm.at[pl.ds(base, B_PER_W)])
        pltpu.sync_copy(irows, iout_hbm.at[pl.ds(base, B_PER_W)])

    return gather_kernel(user_table, item_table, uidx2d, iidx2d)


def _mlp_body(ue_ref, ie_ref, wu1, bu1, wu2, bu2, wi1, bi1, wi2, bi2,
              uo_ref, io_ref):
    u = ue_ref[...]
    hu = jnp.maximum(
        jnp.dot(u, wu1[...], preferred_element_type=jnp.float32) + bu1[...], 0.0)
    uo_ref[...] = jnp.dot(hu, wu2[...], preferred_element_type=jnp.float32) + bu2[...]
    it = ie_ref[...]
    hi = jnp.maximum(
        jnp.dot(it, wi1[...], preferred_element_type=jnp.float32) + bi1[...], 0.0)
    io_ref[...] = jnp.dot(hi, wi2[...], preferred_element_type=jnp.float32) + bi2[...]


def _tc_mlp(ue, ie, Wu1, bu1, Wu2, bu2, Wi1, bi1, Wi2, bi2):
    BM = 2048
    grid = (B // BM,)
    batch_spec = pl.BlockSpec((BM, D), lambda i: (i, 0))
    hid_w = pl.BlockSpec((D, HIDDEN), lambda i: (0, 0))
    out_w = pl.BlockSpec((HIDDEN, D), lambda i: (0, 0))
    hid_b = pl.BlockSpec((1, HIDDEN), lambda i: (0, 0))
    out_b = pl.BlockSpec((1, D), lambda i: (0, 0))
    return pl.pallas_call(
        _mlp_body,
        grid=grid,
        in_specs=[batch_spec, batch_spec,
                  hid_w, hid_b, out_w, out_b,
                  hid_w, hid_b, out_w, out_b],
        out_specs=[batch_spec, batch_spec],
        out_shape=[
            jax.ShapeDtypeStruct((B, D), jnp.float32),
            jax.ShapeDtypeStruct((B, D), jnp.float32),
        ],
    )(ue, ie,
      Wu1, bu1.reshape(1, HIDDEN), Wu2, bu2.reshape(1, D),
      Wi1, bi1.reshape(1, HIDDEN), Wi2, bi2.reshape(1, D))


def kernel(user_input, item_input, user_table, item_table,
           Wu1, bu1, Wu2, bu2, Wi1, bi1, Wi2, bi2):
    uidx2d = user_input.reshape(B // CHUNK, CHUNK)
    iidx2d = item_input.reshape(B // CHUNK, CHUNK)
    ue, ie = _sc_gather(user_table, item_table, uidx2d, iidx2d)
    uo, io = _tc_mlp(ue, ie, Wu1, bu1, Wu2, bu2, Wi1, bi1, Wi2, bi2)
    return (uo, io)
